# Initial kernel scaffold; baseline (speedup 1.0000x reference)
import jax
import jax.numpy as jnp
from jax.experimental import pallas as pl


def kernel(x, mask, fe, fb, w1x, w1f, w1b, b1, w2, b2, wm1, bm1, wm2, bm2, wz1, bz1, wj1z, wj1f, wj1b, bj1, wj2, bj2):
    raise NotImplementedError("write your pallas kernel here")


# speedup vs baseline: 1.0082x; 1.0082x over previous
import functools

import jax
import jax.numpy as jnp
from jax.experimental import pallas as pl
from jax.experimental.pallas import tpu as pltpu

_BF16 = jnp.bfloat16
_F32 = jnp.float32


def _round_up(n, m):
    return ((n + m - 1) // m) * m


def _enc_kernel(x_ref, m_ref, fe_ref, fb_ref,
                w1x_ref, w1f_ref, w1b_ref, b1_ref, w2_ref, b2_ref,
                cpart_ref):
    j = pl.program_id(1)

    @pl.when(j == 0)
    def _init():
        cpart_ref[...] = jnp.zeros_like(cpart_ref)

    x = x_ref[...]
    m = m_ref[...]
    fe = fe_ref[...].astype(_BF16)
    B, T = x.shape

    fl = jnp.dot(fe, w1f_ref[...], preferred_element_type=_F32)
    fl = fl + fb_ref[...] * w1b_ref[...] + b1_ref[...]

    h1 = jnp.maximum(x[:, :, None] * w1x_ref[...][None, :, :]
                     + fl[None, :, :], 0.0)
    Hh = h1.shape[-1]

    h1b = h1.astype(_BF16).reshape(B * T, Hh)
    h2 = jnp.dot(h1b, w2_ref[...], preferred_element_type=_F32) + b2_ref[...]
    h2 = jnp.maximum(h2, 0.0)

    h2 = h2.reshape(B, T, -1) * m[:, :, None]
    cpart_ref[0] += jnp.sum(h2, axis=1)


def _encoder(x, mask, fe, fb, p, tile, steps):
    B = x.shape[0]
    Hh = p['w1f'].shape[1]
    K = p['w2'].shape[1]
    grid = (2, steps)

    full = lambda shape: pl.BlockSpec(shape, lambda i, j: (0, 0))

    c_parts = pl.pallas_call(
        _enc_kernel,
        out_shape=jax.ShapeDtypeStruct((2, B, K), _F32),
        grid_spec=pltpu.PrefetchScalarGridSpec(
            num_scalar_prefetch=0,
            grid=grid,
            in_specs=[
                pl.BlockSpec((B, tile), lambda i, j: (0, i * steps + j)),
                pl.BlockSpec((B, tile), lambda i, j: (0, i * steps + j)),
                pl.BlockSpec((tile, K), lambda i, j: (i * steps + j, 0)),
                pl.BlockSpec((tile, 1), lambda i, j: (i * steps + j, 0)),
                full((1, Hh)),
                full((K, Hh)),
                full((1, Hh)),
                full((1, Hh)),
                full((Hh, K)),
                full((1, K)),
            ],
            out_specs=pl.BlockSpec((1, B, K), lambda i, j: (i, 0, 0)),
        ),
        compiler_params=pltpu.CompilerParams(
            dimension_semantics=("parallel", "arbitrary")),
    )(x, mask, fe, fb,
      p['w1x'], p['w1f'].astype(_BF16), p['w1b'], p['b1'],
      p['w2'].astype(_BF16), p['b2'])
    return c_parts


def _latent_kernel(cp_ref, wm1_ref, bm1_ref, wm2_ref, bm2_ref,
                   wz1_ref, bz1_ref, wj1z_ref,
                   mu_ref, lv_ref, pzp_ref):
    c = cp_ref[0] + cp_ref[1]
    h = jnp.maximum(
        jnp.dot(c, wm1_ref[...], preferred_element_type=_F32) + bm1_ref[...],
        0.0)
    e = jnp.dot(h, wm2_ref[...], preferred_element_type=_F32) + bm2_ref[...]
    L = e.shape[1] // 2
    mu = e[:, :L]
    mu_ref[...] = mu
    lv_ref[...] = e[:, L:]
    pz = jnp.maximum(
        jnp.dot(mu, wz1_ref[...], preferred_element_type=_F32) + bz1_ref[...],
        0.0)
    pzp_ref[...] = jnp.dot(pz, wj1z_ref[...], preferred_element_type=_F32)


def _latent(c_parts, p):
    B = c_parts.shape[1]
    L = p['wz1'].shape[0]
    Hd = p['wz1'].shape[1]
    mu, lv, pzp = pl.pallas_call(
        _latent_kernel,
        out_shape=(jax.ShapeDtypeStruct((B, L), _F32),
                   jax.ShapeDtypeStruct((B, L), _F32),
                   jax.ShapeDtypeStruct((B, Hd), _F32)),
    )(c_parts, p['wm1'], p['bm1'], p['wm2'], p['bm2'],
      p['wz1'], p['bz1'], p['wj1z'])
    return mu, lv, pzp


def _dec_kernel(pzp_ref, fe_ref, fbT_ref,
                wj1fT_ref, wj1bT_ref, bj1T_ref, wj2_ref, bj2_ref,
                rec_ref):
    fe = fe_ref[...].astype(_BF16)
    flT = jax.lax.dot_general(
        wj1fT_ref[...], fe, (((1,), (1,)), ((), ())),
        preferred_element_type=_F32)
    flT = flT + wj1bT_ref[...] * fbT_ref[...] + bj1T_ref[...]
    j1 = jnp.maximum(pzp_ref[...][:, :, None] + flT[None, :, :], 0.0)
    out = jnp.sum(j1 * wj2_ref[...][None, :, :], axis=1)
    rec_ref[...] = out + bj2_ref[0, 0]


def _decoder(pzp, fe, fbT, p, tile, nsteps):
    B, Hd = pzp.shape
    K = fe.shape[1]
    Dp = fe.shape[0]

    full = lambda shape: pl.BlockSpec(shape, lambda i: (0, 0))

    rec = pl.pallas_call(
        _dec_kernel,
        out_shape=jax.ShapeDtypeStruct((B, Dp), _F32),
        grid_spec=pltpu.PrefetchScalarGridSpec(
            num_scalar_prefetch=0,
            grid=(nsteps,),
            in_specs=[
                full((B, Hd)),
                pl.BlockSpec((tile, K), lambda i: (i, 0)),
                pl.BlockSpec((1, tile), lambda i: (0, i)),
                full((Hd, K)),
                full((Hd, 1)),
                full((Hd, 1)),
                full((Hd, 1)),
                full((1, 1)),
            ],
            out_specs=pl.BlockSpec((B, tile), lambda i: (0, i)),
        ),
        compiler_params=pltpu.CompilerParams(
            dimension_semantics=("parallel",)),
    )(pzp, fe, fbT,
      p['wj1f'].T.astype(_BF16), p['wj1b'].T, p['bj1'].T, p['wj2'], p['bj2'])
    return rec


@functools.partial(jax.jit, static_argnames=("tile",))
def _forward(x, mask, p, *, tile=512):
    B, D = x.shape
    chunk = 2 * tile
    Dp = _round_up(D, chunk)
    fe, fb = p['fe'], p['fb']

    pad = Dp - D
    if pad:
        x = jnp.pad(x, ((0, 0), (0, pad)))
        mask = jnp.pad(mask, ((0, 0), (0, pad)))
        fe = jnp.pad(fe, ((0, pad), (0, 0)))
        fb = jnp.pad(fb, ((0, pad), (0, 0)))

    steps = Dp // chunk
    c_parts = _encoder(x, mask, fe, fb, p, tile, steps)
    mu, logvar, pzp = _latent(c_parts, p)
    rec = _decoder(pzp, fe, fb.T, p, tile, Dp // tile)
    return rec[:, :D], mu, logvar


def kernel(x, mask, fe, fb, w1x, w1f, w1b, b1, w2, b2, wm1, bm1, wm2, bm2,
           wz1, bz1, wj1z, wj1f, wj1b, bj1, wj2, bj2):
    p = {
        "fe": fe, "fb": fb, "w1x": w1x, "w1f": w1f, "w1b": w1b, "b1": b1,
        "w2": w2, "b2": b2, "wm1": wm1, "bm1": bm1, "wm2": wm2, "bm2": bm2,
        "wz1": wz1, "bz1": bz1, "wj1z": wj1z, "wj1f": wj1f, "wj1b": wj1b,
        "bj1": bj1, "wj2": wj2, "bj2": bj2,
    }
    return _forward(x, mask, p, tile=512)


# speedup vs baseline: 1.7106x; 1.6967x over previous
import functools

import jax
import jax.numpy as jnp
from jax.experimental import pallas as pl
from jax.experimental.pallas import tpu as pltpu

_BF16 = jnp.bfloat16
_F32 = jnp.float32


def _round_up(n, m):
    return ((n + m - 1) // m) * m


def _enc_kernel(x_ref, m_ref, fe_ref, fbT_ref,
                w1xT_ref, w1f_ref, w1bT_ref, b1T_ref, w2aug_ref,
                cpart_ref):
    j = pl.program_id(1)

    @pl.when(j == 0)
    def _init():
        cpart_ref[...] = jnp.zeros_like(cpart_ref)

    flT = jax.lax.dot_general(
        w1f_ref[...], fe_ref[...].astype(_BF16),
        (((0,), (1,)), ((), ())), preferred_element_type=_F32)
    flT = flT + w1bT_ref[...] * fbT_ref[...] + b1T_ref[...]
    flTb = flT.astype(_BF16)

    xb = x_ref[...].astype(_BF16)
    mb = m_ref[...].astype(_BF16)

    h1 = jnp.maximum(xb[:, None, :] * w1xT_ref[...][None, :, :]
                     + flTb[None, :, :], jnp.bfloat16(0))
    h1m = h1 * mb[:, None, :]
    h1aug = jnp.concatenate([h1m, mb[:, None, :]], axis=1)

    z = jax.lax.dot_general(
        h1aug, w2aug_ref[...],
        (((1,), (0,)), ((), ())), preferred_element_type=_F32)

    cpart_ref[0] += jnp.sum(jnp.maximum(z, 0.0), axis=1)


def _encoder(x, mask, fe, fbT, p, tile, steps):
    B = x.shape[0]
    Hh = p['w1f'].shape[1]
    K = p['w2'].shape[1]
    grid = (2, steps)

    w2aug = jnp.concatenate([p['w2'], p['b2']], axis=0).astype(_BF16)

    full = lambda shape: pl.BlockSpec(shape, lambda i, j: (0, 0))

    c_parts = pl.pallas_call(
        _enc_kernel,
        out_shape=jax.ShapeDtypeStruct((2, B, K), _F32),
        grid_spec=pltpu.PrefetchScalarGridSpec(
            num_scalar_prefetch=0,
            grid=grid,
            in_specs=[
                pl.BlockSpec((B, tile), lambda i, j: (0, i * steps + j)),
                pl.BlockSpec((B, tile), lambda i, j: (0, i * steps + j)),
                pl.BlockSpec((tile, K), lambda i, j: (i * steps + j, 0)),
                pl.BlockSpec((1, tile), lambda i, j: (0, i * steps + j)),
                full((Hh, 1)),
                full((K, Hh)),
                full((Hh, 1)),
                full((Hh, 1)),
                full((Hh + 1, K)),
            ],
            out_specs=pl.BlockSpec((1, B, K), lambda i, j: (i, 0, 0)),
        ),
        compiler_params=pltpu.CompilerParams(
            dimension_semantics=("parallel", "arbitrary")),
    )(x, mask, fe, fbT,
      p['w1x'].T.astype(_BF16), p['w1f'].astype(_BF16),
      p['w1b'].T, p['b1'].T, w2aug)
    return c_parts


def _latent_kernel(cp_ref, wm1_ref, bm1_ref, wm2_ref, bm2_ref,
                   wz1_ref, bz1_ref, wj1z_ref,
                   mu_ref, lv_ref, pzp_ref):
    c = cp_ref[0] + cp_ref[1]
    h = jnp.maximum(
        jnp.dot(c, wm1_ref[...], preferred_element_type=_F32) + bm1_ref[...],
        0.0)
    e = jnp.dot(h, wm2_ref[...], preferred_element_type=_F32) + bm2_ref[...]
    L = e.shape[1] // 2
    mu = e[:, :L]
    mu_ref[...] = mu
    lv_ref[...] = e[:, L:]
    pz = jnp.maximum(
        jnp.dot(mu, wz1_ref[...], preferred_element_type=_F32) + bz1_ref[...],
        0.0)
    pzp_ref[...] = jnp.dot(pz, wj1z_ref[...], preferred_element_type=_F32)


def _latent(c_parts, p):
    B = c_parts.shape[1]
    L = p['wz1'].shape[0]
    Hd = p['wz1'].shape[1]
    mu, lv, pzp = pl.pallas_call(
        _latent_kernel,
        out_shape=(jax.ShapeDtypeStruct((B, L), _F32),
                   jax.ShapeDtypeStruct((B, L), _F32),
                   jax.ShapeDtypeStruct((B, Hd), _F32)),
    )(c_parts, p['wm1'], p['bm1'], p['wm2'], p['bm2'],
      p['wz1'], p['bz1'], p['wj1z'])
    return mu, lv, pzp


def _dec_kernel(pzp_ref, fe_ref, fbT_ref,
                wj1f_ref, wj1bT_ref, bj1T_ref, wj2_ref, bj2_ref,
                rec_ref):
    flT = jax.lax.dot_general(
        wj1f_ref[...], fe_ref[...].astype(_BF16),
        (((0,), (1,)), ((), ())), preferred_element_type=_F32)
    flT = flT + wj1bT_ref[...] * fbT_ref[...] + bj1T_ref[...]
    flTb = flT.astype(_BF16)

    pzpb = pzp_ref[...].astype(_BF16)
    j1 = jnp.maximum(pzpb[:, :, None] + flTb[None, :, :],
                     jnp.bfloat16(0))
    prod = (j1 * wj2_ref[...].astype(_BF16)[None, :, :]).astype(_F32)
    rec_ref[...] = jnp.sum(prod, axis=1) + bj2_ref[0, 0]


def _decoder(pzp, fe, fbT, p, tile, nsteps):
    B, Hd = pzp.shape
    K = fe.shape[1]
    Dp = fe.shape[0]

    full = lambda shape: pl.BlockSpec(shape, lambda i: (0, 0))

    rec = pl.pallas_call(
        _dec_kernel,
        out_shape=jax.ShapeDtypeStruct((B, Dp), _F32),
        grid_spec=pltpu.PrefetchScalarGridSpec(
            num_scalar_prefetch=0,
            grid=(nsteps,),
            in_specs=[
                full((B, Hd)),
                pl.BlockSpec((tile, K), lambda i: (i, 0)),
                pl.BlockSpec((1, tile), lambda i: (0, i)),
                full((K, Hd)),
                full((Hd, 1)),
                full((Hd, 1)),
                full((Hd, 1)),
                full((1, 1)),
            ],
            out_specs=pl.BlockSpec((B, tile), lambda i: (0, i)),
        ),
        compiler_params=pltpu.CompilerParams(
            dimension_semantics=("parallel",)),
    )(pzp, fe, fbT,
      p['wj1f'].astype(_BF16), p['wj1b'].T, p['bj1'].T, p['wj2'], p['bj2'])
    return rec


@functools.partial(jax.jit, static_argnames=("tile",))
def _forward(x, mask, p, *, tile=512):
    B, D = x.shape
    chunk = 2 * tile
    Dp = _round_up(D, chunk)
    fe, fb = p['fe'], p['fb']

    pad = Dp - D
    if pad:
        x = jnp.pad(x, ((0, 0), (0, pad)))
        mask = jnp.pad(mask, ((0, 0), (0, pad)))
        fe = jnp.pad(fe, ((0, pad), (0, 0)))
        fb = jnp.pad(fb, ((0, pad), (0, 0)))

    fbT = fb.T
    steps = Dp // chunk
    c_parts = _encoder(x, mask, fe, fbT, p, tile, steps)
    mu, logvar, pzp = _latent(c_parts, p)
    rec = _decoder(pzp, fe, fbT, p, tile, Dp // tile)
    return rec[:, :D], mu, logvar


def kernel(x, mask, fe, fb, w1x, w1f, w1b, b1, w2, b2, wm1, bm1, wm2, bm2,
           wz1, bz1, wj1z, wj1f, wj1b, bj1, wj2, bj2):
    p = {
        "fe": fe, "fb": fb, "w1x": w1x, "w1f": w1f, "w1b": w1b, "b1": b1,
        "w2": w2, "b2": b2, "wm1": wm1, "bm1": bm1, "wm2": wm2, "bm2": bm2,
        "wz1": wz1, "bz1": bz1, "wj1z": wj1z, "wj1f": wj1f, "wj1b": wj1b,
        "bj1": bj1, "wj2": wj2, "bj2": bj2,
    }
    return _forward(x, mask, p, tile=512)
